# baseline (speedup 1.0000x reference)
import functools

import jax
import jax.numpy as jnp
from jax import lax
from jax.experimental import pallas as pl
from jax.experimental.pallas import tpu as pltpu
from jax.experimental.pallas import tpu_sc as plsc

BATCH = 16384
EMBED_DIM = 64
NUM_CORES = 2
NUM_SUBCORES = 16
NUM_WORKERS = NUM_CORES * NUM_SUBCORES
BPW = BATCH // NUM_WORKERS
CHUNK = 128
NCHUNK = BPW // CHUNK
LANES = 16
NGROUP = BPW // LANES


def _sc_body(users_hbm, items_hbm, uemb_hbm, iemb_hbm, ub_hbm, ib_hbm,
             gb_hbm, out_hbm,
             uidx_v, iidx_v, urows_v, irows_v, ubias_v, ibias_v, gb_v,
             out_v, sem):
    wid = lax.axis_index("s") * NUM_CORES + lax.axis_index("c")
    base = wid * BPW

    for j in range(NCHUNK):
        pltpu.sync_copy(users_hbm.at[pl.ds(base + j * CHUNK, CHUNK)],
                        uidx_v.at[j])
        pltpu.sync_copy(items_hbm.at[pl.ds(base + j * CHUNK, CHUNK)],
                        iidx_v.at[j])
    pltpu.sync_copy(gb_hbm, gb_v)

    copies = []
    for j in range(NCHUNK):
        sl = pl.ds(j * CHUNK, CHUNK)
        copies.append(pltpu.async_copy(uemb_hbm.at[uidx_v.at[j]],
                                       urows_v.at[sl], sem))
        copies.append(pltpu.async_copy(iemb_hbm.at[iidx_v.at[j]],
                                       irows_v.at[sl], sem))
        copies.append(pltpu.async_copy(ub_hbm.at[uidx_v.at[j]],
                                       ubias_v.at[sl], sem))
        copies.append(pltpu.async_copy(ib_hbm.at[iidx_v.at[j]],
                                       ibias_v.at[sl], sem))
    for c in copies:
        c.wait()

    iota = jnp.arange(LANES, dtype=jnp.int32)
    zeros_i = jnp.zeros((LANES,), dtype=jnp.int32)
    gbias = gb_v[...]

    def group_body(g, _):
        rows = g * LANES + iota
        acc = jnp.zeros((LANES,), dtype=jnp.float32)
        for d in range(EMBED_DIM):
            col = jnp.full((LANES,), d, dtype=jnp.int32)
            cu = plsc.load_gather(urows_v, [rows, col])
            ci = plsc.load_gather(irows_v, [rows, col])
            acc = acc + cu * ci
        ub = ubias_v[pl.ds(g * LANES, LANES)]
        ib = ibias_v[pl.ds(g * LANES, LANES)]
        out_v[pl.ds(g * LANES, LANES)] = acc + ub + ib + gbias
        return 0

    lax.fori_loop(0, NGROUP, group_body, 0)

    pltpu.sync_copy(out_v, out_hbm.at[pl.ds(base, BPW)])


@jax.jit
def _bpr_scores(users, items, user_emb_w, item_emb_w, user_bias_w,
                item_bias_w, global_bias):
    mesh = plsc.VectorSubcoreMesh(core_axis_name="c", subcore_axis_name="s",
                                  num_cores=NUM_CORES,
                                  num_subcores=NUM_SUBCORES)
    f = pl.kernel(
        _sc_body,
        out_type=jax.ShapeDtypeStruct((BATCH,), jnp.float32),
        mesh=mesh,
        compiler_params=pltpu.CompilerParams(needs_layout_passes=False,
                                             use_tc_tiling_on_sc=False),
        scratch_types=[
            pltpu.VMEM((NCHUNK, CHUNK), jnp.int32),
            pltpu.VMEM((NCHUNK, CHUNK), jnp.int32),
            pltpu.VMEM((BPW, EMBED_DIM), jnp.float32),
            pltpu.VMEM((BPW, EMBED_DIM), jnp.float32),
            pltpu.VMEM((BPW,), jnp.float32),
            pltpu.VMEM((BPW,), jnp.float32),
            pltpu.VMEM((LANES,), jnp.float32),
            pltpu.VMEM((BPW,), jnp.float32),
            pltpu.SemaphoreType.DMA,
        ],
    )
    return f(users, items, user_emb_w, item_emb_w, user_bias_w, item_bias_w,
             global_bias)


def kernel(users, items, user_emb_w, item_emb_w, user_bias_w, item_bias_w,
           global_bias):
    users = users.astype(jnp.int32)
    items = items.astype(jnp.int32)
    gb16 = jnp.broadcast_to(global_bias.reshape(()), (16,))
    return _bpr_scores(users, items, user_emb_w, item_emb_w,
                       user_bias_w.reshape(-1), item_bias_w.reshape(-1),
                       gb16)
